# baseline (speedup 1.0000x reference)
import functools

import jax
import jax.numpy as jnp
from jax import lax
from jax.experimental import pallas as pl
from jax.experimental.pallas import tpu as pltpu
from jax.experimental.pallas import tpu_sc as plsc

CHUNK = 8
PAIR = 4
NBUF = 8


@functools.lru_cache(maxsize=None)
def _build(batch: int, seq: int, d_model: int):
    info = plsc.get_sparse_core_info()
    nc, ns = info.num_cores, info.num_subcores
    nw = nc * ns
    n_tokens = batch * seq
    assert n_tokens % (nw * CHUNK) == 0 and seq % CHUNK == 0
    bpw = n_tokens // nw
    wpr = seq // bpw
    nchunks = bpw // CHUNK
    npairs = NBUF // PAIR
    assert nchunks % NBUF == 0
    mesh = plsc.VectorSubcoreMesh(core_axis_name="c", subcore_axis_name="s")

    @functools.partial(
        pl.kernel,
        mesh=mesh,
        out_type=jax.ShapeDtypeStruct((n_tokens, d_model), jnp.float32),
        scratch_types=(
            [pltpu.VMEM((bpw,), jnp.int32),
             pltpu.VMEM((NBUF * CHUNK, d_model), jnp.float32)]
            + [pltpu.SemaphoreType.DMA for _ in range(NBUF + npairs)]
        ),
    )
    def emb(table_hbm, idx_hbm, out_hbm, idx_v, big, *sems):
        gsems = sems[:NBUF]
        psems = sems[NBUF:]
        wid = lax.axis_index("s") * nc + lax.axis_index("c")
        base = wid * bpw
        pltpu.sync_copy(
            idx_hbm.at[wid // wpr, pl.ds((wid % wpr) * bpw, bpw)], idx_v)

        def start_gather(g, b):
            off = pl.multiple_of(g * CHUNK, CHUNK)
            pltpu.async_copy(
                table_hbm.at[idx_v.at[pl.ds(off, CHUNK)]],
                big.at[pl.ds(b * CHUNK, CHUNK)], gsems[b])

        def wait_gather(b):
            pltpu.make_async_copy(
                table_hbm.at[idx_v.at[pl.ds(0, CHUNK)]],
                big.at[pl.ds(0, CHUNK)], gsems[b]).wait()

        def start_store(g, p):
            off = pl.multiple_of(base + g * CHUNK, CHUNK)
            pltpu.async_copy(
                big.at[pl.ds(p * PAIR * CHUNK, PAIR * CHUNK)],
                out_hbm.at[pl.ds(off, PAIR * CHUNK)], psems[p])

        def wait_store(p):
            pltpu.make_async_copy(
                big.at[pl.ds(0, PAIR * CHUNK)],
                out_hbm.at[pl.ds(0, PAIR * CHUNK)], psems[p]).wait()

        for b in range(NBUF):
            start_gather(b, b)

        def body(o, _):
            g0 = o * NBUF
            for p in range(npairs):
                g = g0 + p * PAIR
                for j in range(PAIR):
                    wait_gather(p * PAIR + j)
                start_store(g, p)
                ng = g + NBUF

                @pl.when(ng < nchunks)
                def _():
                    wait_store(p)
                    for j in range(PAIR):
                        start_gather(ng + j, p * PAIR + j)

            return 0

        lax.fori_loop(0, nchunks // NBUF, body, 0)
        for p in range(npairs):
            wait_store(p)

    return emb


def kernel(token_ids, weight):
    batch, seq = token_ids.shape
    out = _build(batch, seq, weight.shape[1])(
        weight, token_ids.astype(jnp.int32))
    return out.reshape(batch, seq, weight.shape[1])


# speedup vs baseline: 1.0132x; 1.0132x over previous
import functools

import jax
import jax.numpy as jnp
from jax import lax
from jax.experimental import pallas as pl
from jax.experimental.pallas import tpu as pltpu
from jax.experimental.pallas import tpu_sc as plsc

CHUNK = 8
PAIR = 2
NBUF = 8


@functools.lru_cache(maxsize=None)
def _build(batch: int, seq: int, d_model: int):
    info = plsc.get_sparse_core_info()
    nc, ns = info.num_cores, info.num_subcores
    nw = nc * ns
    n_tokens = batch * seq
    assert n_tokens % (nw * CHUNK) == 0 and seq % CHUNK == 0
    bpw = n_tokens // nw
    wpr = seq // bpw
    nchunks = bpw // CHUNK
    npairs = NBUF // PAIR
    assert nchunks % NBUF == 0
    mesh = plsc.VectorSubcoreMesh(core_axis_name="c", subcore_axis_name="s")

    @functools.partial(
        pl.kernel,
        mesh=mesh,
        out_type=jax.ShapeDtypeStruct((n_tokens, d_model), jnp.float32),
        scratch_types=(
            [pltpu.VMEM((bpw,), jnp.int32),
             pltpu.VMEM((NBUF * CHUNK, d_model), jnp.float32)]
            + [pltpu.SemaphoreType.DMA for _ in range(NBUF + npairs)]
        ),
    )
    def emb(table_hbm, idx_hbm, out_hbm, idx_v, big, *sems):
        gsems = sems[:NBUF]
        psems = sems[NBUF:]
        wid = lax.axis_index("s") * nc + lax.axis_index("c")
        base = wid * bpw
        pltpu.sync_copy(
            idx_hbm.at[wid // wpr, pl.ds((wid % wpr) * bpw, bpw)], idx_v)

        def start_gather(g, b):
            off = pl.multiple_of(g * CHUNK, CHUNK)
            pltpu.async_copy(
                table_hbm.at[idx_v.at[pl.ds(off, CHUNK)]],
                big.at[pl.ds(b * CHUNK, CHUNK)], gsems[b])

        def wait_gather(b):
            pltpu.make_async_copy(
                table_hbm.at[idx_v.at[pl.ds(0, CHUNK)]],
                big.at[pl.ds(0, CHUNK)], gsems[b]).wait()

        def start_store(g, p):
            off = pl.multiple_of(base + g * CHUNK, CHUNK)
            pltpu.async_copy(
                big.at[pl.ds(p * PAIR * CHUNK, PAIR * CHUNK)],
                out_hbm.at[pl.ds(off, PAIR * CHUNK)], psems[p])

        def wait_store(p):
            pltpu.make_async_copy(
                big.at[pl.ds(0, PAIR * CHUNK)],
                out_hbm.at[pl.ds(0, PAIR * CHUNK)], psems[p]).wait()

        for b in range(NBUF):
            start_gather(b, b)

        def body(o, _):
            g0 = o * NBUF
            for p in range(npairs):
                g = g0 + p * PAIR
                for j in range(PAIR):
                    wait_gather(p * PAIR + j)
                start_store(g, p)
                ng = g + NBUF

                @pl.when(ng < nchunks)
                def _():
                    wait_store(p)
                    for j in range(PAIR):
                        start_gather(ng + j, p * PAIR + j)

            return 0

        lax.fori_loop(0, nchunks // NBUF, body, 0)
        for p in range(npairs):
            wait_store(p)

    return emb


def kernel(token_ids, weight):
    batch, seq = token_ids.shape
    out = _build(batch, seq, weight.shape[1])(
        weight, token_ids.astype(jnp.int32))
    return out.reshape(batch, seq, weight.shape[1])
